# baseline (speedup 1.0000x reference)
import functools

import jax
import jax.numpy as jnp
import numpy as np
from jax import lax
from jax.experimental import pallas as pl
from jax.experimental.pallas import tpu as pltpu
from jax.experimental.pallas import tpu_sc as plsc

B, N, E, DN, DE, D = 4, 10000, 160000, 128, 16, 128

NTILE = 16
C = 80
EPT = E // NTILE
NCHUNK = EPT // C
RPT = 624
TAIL = N - NTILE * RPT
ZR = 16


def _linear_body(x_ref, w_ref, b_ref, o_ref, *, relu):
    y = jnp.dot(x_ref[...], w_ref[...], preferred_element_type=jnp.float32)
    y = y + b_ref[...]
    if relu:
        y = jnp.maximum(y, 0.0)
    o_ref[...] = y.astype(o_ref.dtype)


def _linear(x, w, b, relu=False, block_rows=5000, out_dtype=jnp.float32):
    R, K = x.shape
    Dout = w.shape[1]
    return pl.pallas_call(
        functools.partial(_linear_body, relu=relu),
        grid=(R // block_rows,),
        in_specs=[
            pl.BlockSpec((block_rows, K), lambda i: (i, 0)),
            pl.BlockSpec((K, Dout), lambda i: (0, 0)),
            pl.BlockSpec((1, Dout), lambda i: (0, 0)),
        ],
        out_specs=pl.BlockSpec((block_rows, Dout), lambda i: (i, 0)),
        out_shape=jax.ShapeDtypeStruct((R, Dout), out_dtype),
    )(x, w, b.reshape(1, Dout))


def _linear_pack_body(x_ref, w_ref, b_ref, o_ref):
    y = jnp.dot(x_ref[...], w_ref[...], preferred_element_type=jnp.float32)
    y = y + b_ref[...]
    bits = lax.bitcast_convert_type(y, jnp.int32) + 0x8000
    lo = lax.shift_right_logical(bits[:, : y.shape[1] // 2], 16)
    hi = bits[:, y.shape[1] // 2:] & jnp.int32(-65536)
    o_ref[...] = lo | hi


def _linear_pack(x, w, b, block_rows=5000):
    R, K = x.shape
    Dout = w.shape[1]
    return pl.pallas_call(
        _linear_pack_body,
        grid=(R // block_rows,),
        in_specs=[
            pl.BlockSpec((block_rows, K), lambda i: (i, 0)),
            pl.BlockSpec((K, Dout), lambda i: (0, 0)),
            pl.BlockSpec((1, Dout), lambda i: (0, 0)),
        ],
        out_specs=pl.BlockSpec((block_rows, Dout // 2), lambda i: (i, 0)),
        out_shape=jax.ShapeDtypeStruct((R, Dout // 2), jnp.int32),
    )(x, w, b.reshape(1, Dout))


NBUF = 2


def _sc_message_pass(xenc, ei_flat, eenc, gi):
    mesh = plsc.VectorSubcoreMesh(core_axis_name="c", subcore_axis_name="s")

    @functools.partial(
        pl.kernel,
        out_type=jax.ShapeDtypeStruct((2 * N, D), jnp.float32),
        mesh=mesh,
        scratch_types=[
            pltpu.VMEM((NBUF, C), jnp.int32),
            pltpu.VMEM((NBUF, C), jnp.int32),
            pltpu.VMEM((NBUF, C, D), jnp.float32),
            pltpu.VMEM((NBUF, C, D // 2), jnp.int32),
            pltpu.VMEM((ZR, D), jnp.float32),
            pltpu.VMEM_SHARED((N, D), jnp.float32),
        ]
        + [pltpu.SemaphoreType.DMA] * (3 * NBUF),
    )
    def k(xenc_hbm, ei_hbm, eenc_hbm, out_hbm,
          gidx, didx, rows, embuf, zbuf, agg, *sems):
        sem_i = sems[0:NBUF]
        sem_g = sems[NBUF:2 * NBUF]
        sem_e = sems[2 * NBUF:3 * NBUF]
        c = lax.axis_index("c")
        s = lax.axis_index("s")
        zvec = jnp.zeros((16,), jnp.float32)

        def zero_body(i, _):
            for j in range(D // 16):
                zbuf[i, pl.ds(j * 16, 16)] = zvec
            return 0
        lax.fori_loop(0, ZR, zero_body, 0)
        r0 = s * RPT

        def zero_agg():
            for kk in range(RPT // ZR):
                pltpu.sync_copy(zbuf, agg.at[pl.ds(r0 + kk * ZR, ZR)])
            @pl.when(s == NTILE - 1)
            def _():
                pltpu.sync_copy(zbuf.at[pl.ds(0, TAIL)],
                                agg.at[pl.ds(NTILE * RPT, TAIL)])

        zero_agg()

        if True:
            lc = c
            tile0 = s * EPT
            sb = lc * 2 * E + tile0
            db = sb + E
            eb = lc * E + tile0
            bnv = (2 * gi + c) * N + jnp.zeros((16,), jnp.int32)

            def start_idx(j, bb):
                pltpu.async_copy(ei_hbm.at[pl.ds(sb + j * C, C)],
                                 gidx.at[bb], sem_i[bb])
                pltpu.async_copy(ei_hbm.at[pl.ds(db + j * C, C)],
                                 didx.at[bb], sem_i[bb])

            def start_ge(j, bb):
                pltpu.make_async_copy(ei_hbm.at[pl.ds(sb + j * C, C)],
                                      gidx.at[bb], sem_i[bb]).wait()
                pltpu.make_async_copy(ei_hbm.at[pl.ds(db + j * C, C)],
                                      didx.at[bb], sem_i[bb]).wait()
                for kk in range(C // 16):
                    sl = pl.ds(kk * 16, 16)
                    gidx[bb, sl] = gidx[bb, sl] + bnv
                pltpu.async_copy(xenc_hbm.at[gidx.at[bb]], rows.at[bb],
                                 sem_g[bb])
                pltpu.async_copy(eenc_hbm.at[pl.ds(eb + j * C, C)],
                                 embuf.at[bb], sem_e[bb])

            def phase(j, bb):
                nb = (bb + 1) % NBUF
                @pl.when(j + 1 < NCHUNK)
                def _():
                    start_ge(j + 1, nb)
                pltpu.make_async_copy(xenc_hbm.at[gidx.at[bb]],
                                      rows.at[bb], sem_g[bb]).wait()
                pltpu.make_async_copy(eenc_hbm.at[pl.ds(eb + j * C, C)],
                                      embuf.at[bb], sem_e[bb]).wait()

                himask = jnp.full((16,), -65536, jnp.int32)

                def mul_body(i, _):
                    for jj in range(D // 32):
                        ew = embuf[bb, i, pl.ds(jj * 16, 16)]
                        elo = lax.bitcast_convert_type(ew << 16, jnp.float32)
                        ehi = lax.bitcast_convert_type(ew & himask, jnp.float32)
                        sl_lo = pl.ds(jj * 32, 16)
                        sl_hi = pl.ds(jj * 32 + 16, 16)
                        rows[bb, i, sl_lo] = rows[bb, i, sl_lo] * elo
                        rows[bb, i, sl_hi] = rows[bb, i, sl_hi] * ehi
                    return 0
                lax.fori_loop(0, C, mul_body, 0)
                pltpu.sync_copy(rows.at[bb], agg.at[didx.at[bb]], add=True)
                @pl.when(j + 2 < NCHUNK)
                def _():
                    start_idx(j + 2, bb)

            plsc.subcore_barrier()

            start_idx(0, 0)
            start_ge(0, 0)
            start_idx(1, 1)

            def pair_body(jj, _):
                for ph in range(NBUF):
                    phase(jj * NBUF + ph, ph)
                return 0
            lax.fori_loop(0, NCHUNK // NBUF, pair_body, 0)
            for j in range(NCHUNK - NCHUNK % NBUF, NCHUNK):
                phase(jnp.int32(j), j % NBUF)

            plsc.subcore_barrier()
            pltpu.sync_copy(agg.at[pl.ds(r0, RPT)],
                            out_hbm.at[pl.ds(lc * N + r0, RPT)])
            @pl.when(s == NTILE - 1)
            def _():
                pltpu.sync_copy(agg.at[pl.ds(NTILE * RPT, TAIL)],
                                out_hbm.at[pl.ds(lc * N + NTILE * RPT, TAIL)])

    return k(xenc, ei_flat, eenc)


_Q = np.concatenate([np.concatenate([np.arange(16 * j, 16 * j + 16),
                                     64 + np.arange(16 * j, 16 * j + 16)])
                     for j in range(D // 32)])


def kernel(x, edge_index, edge_attr, W_node, b_node, W_edge, b_edge, W_out, b_out):
    xenc = _linear(x.reshape(B * N, DN), W_node[:, _Q], b_node[_Q])
    ea = edge_attr.reshape(B * E, DE)
    w_out_q = W_out[_Q, :]
    outs = []
    for gi in range(2):
        eenc_i = _linear_pack(ea[gi * 2 * E:(gi + 1) * 2 * E], W_edge, b_edge)
        agg = _sc_message_pass(xenc,
                               edge_index[2 * gi:2 * gi + 2].reshape(-1),
                               eenc_i, gi)
        outs.append(_linear(agg, w_out_q, b_out, relu=True))
    return jnp.concatenate(outs, axis=0).reshape(B, N, D)


# speedup vs baseline: 1.0565x; 1.0565x over previous
import functools

import jax
import jax.numpy as jnp
import numpy as np
from jax import lax
from jax.experimental import pallas as pl
from jax.experimental.pallas import tpu as pltpu
from jax.experimental.pallas import tpu_sc as plsc

B, N, E, DN, DE, D = 4, 10000, 160000, 128, 16, 128

NTILE = 16
C = 80
EPT = E // NTILE
NCHUNK = EPT // C
RPT = 624
TAIL = N - NTILE * RPT
ZR = 16


def _linear_body(x_ref, w_ref, b_ref, o_ref, *, relu):
    y = jnp.dot(x_ref[...], w_ref[...], preferred_element_type=jnp.float32)
    y = y + b_ref[...]
    if relu:
        y = jnp.maximum(y, 0.0)
    o_ref[...] = y.astype(o_ref.dtype)


def _linear(x, w, b, relu=False, block_rows=5000, out_dtype=jnp.float32):
    R, K = x.shape
    Dout = w.shape[1]
    return pl.pallas_call(
        functools.partial(_linear_body, relu=relu),
        grid=(R // block_rows,),
        in_specs=[
            pl.BlockSpec((block_rows, K), lambda i: (i, 0)),
            pl.BlockSpec((K, Dout), lambda i: (0, 0)),
            pl.BlockSpec((1, Dout), lambda i: (0, 0)),
        ],
        out_specs=pl.BlockSpec((block_rows, Dout), lambda i: (i, 0)),
        out_shape=jax.ShapeDtypeStruct((R, Dout), out_dtype),
    )(x, w, b.reshape(1, Dout))


def _linear_pack_body(x_ref, w_ref, b_ref, o_ref):
    y = jnp.dot(x_ref[...], w_ref[...], preferred_element_type=jnp.float32)
    y = y + b_ref[...]
    bits = lax.bitcast_convert_type(y, jnp.int32) + 0x8000
    lo = lax.shift_right_logical(bits[:, : y.shape[1] // 2], 16)
    hi = bits[:, y.shape[1] // 2:] & jnp.int32(-65536)
    o_ref[...] = lo | hi


def _linear_pack(x, w, b, block_rows=5000):
    R, K = x.shape
    Dout = w.shape[1]
    return pl.pallas_call(
        _linear_pack_body,
        grid=(R // block_rows,),
        in_specs=[
            pl.BlockSpec((block_rows, K), lambda i: (i, 0)),
            pl.BlockSpec((K, Dout), lambda i: (0, 0)),
            pl.BlockSpec((1, Dout), lambda i: (0, 0)),
        ],
        out_specs=pl.BlockSpec((block_rows, Dout // 2), lambda i: (i, 0)),
        out_shape=jax.ShapeDtypeStruct((R, Dout // 2), jnp.int32),
    )(x, w, b.reshape(1, Dout))


NBUF = 2


def _sc_message_pass(xenc, ei_flat, eenc):
    mesh = plsc.VectorSubcoreMesh(core_axis_name="c", subcore_axis_name="s")

    @functools.partial(
        pl.kernel,
        out_type=jax.ShapeDtypeStruct((B * N, D), jnp.float32),
        mesh=mesh,
        scratch_types=[
            pltpu.VMEM((NBUF, C), jnp.int32),
            pltpu.VMEM((NBUF, C), jnp.int32),
            pltpu.VMEM((NBUF, C, D), jnp.float32),
            pltpu.VMEM((NBUF, C, D // 2), jnp.int32),
            pltpu.VMEM((ZR, D), jnp.float32),
            pltpu.VMEM_SHARED((N, D), jnp.float32),
        ]
        + [pltpu.SemaphoreType.DMA] * (3 * NBUF),
    )
    def k(xenc_hbm, ei_hbm, eenc_hbm, out_hbm,
          gidx, didx, rows, embuf, zbuf, agg, *sems):
        sem_i = sems[0:NBUF]
        sem_g = sems[NBUF:2 * NBUF]
        sem_e = sems[2 * NBUF:3 * NBUF]
        c = lax.axis_index("c")
        s = lax.axis_index("s")
        zvec = jnp.zeros((16,), jnp.float32)

        def zero_body(i, _):
            for j in range(D // 16):
                zbuf[i, pl.ds(j * 16, 16)] = zvec
            return 0
        lax.fori_loop(0, ZR, zero_body, 0)
        r0 = s * RPT

        def zero_agg():
            for kk in range(RPT // ZR):
                pltpu.sync_copy(zbuf, agg.at[pl.ds(r0 + kk * ZR, ZR)])
            @pl.when(s == NTILE - 1)
            def _():
                pltpu.sync_copy(zbuf.at[pl.ds(0, TAIL)],
                                agg.at[pl.ds(NTILE * RPT, TAIL)])

        zero_agg()

        for g in range(2):
            b = 2 * c + g
            tile0 = s * EPT
            sb = b * 2 * E + tile0
            db = sb + E
            eb = b * E + tile0
            bnv = b * N + jnp.zeros((16,), jnp.int32)

            def start_idx(j, bb):
                pltpu.async_copy(ei_hbm.at[pl.ds(sb + j * C, C)],
                                 gidx.at[bb], sem_i[bb])
                pltpu.async_copy(ei_hbm.at[pl.ds(db + j * C, C)],
                                 didx.at[bb], sem_i[bb])

            def start_ge(j, bb):
                pltpu.make_async_copy(ei_hbm.at[pl.ds(sb + j * C, C)],
                                      gidx.at[bb], sem_i[bb]).wait()
                pltpu.make_async_copy(ei_hbm.at[pl.ds(db + j * C, C)],
                                      didx.at[bb], sem_i[bb]).wait()
                for kk in range(C // 16):
                    sl = pl.ds(kk * 16, 16)
                    gidx[bb, sl] = gidx[bb, sl] + bnv
                pltpu.async_copy(xenc_hbm.at[gidx.at[bb]], rows.at[bb],
                                 sem_g[bb])
                pltpu.async_copy(eenc_hbm.at[pl.ds(eb + j * C, C)],
                                 embuf.at[bb], sem_e[bb])

            def phase(j, bb):
                nb = (bb + 1) % NBUF
                @pl.when(j + 1 < NCHUNK)
                def _():
                    start_ge(j + 1, nb)
                pltpu.make_async_copy(xenc_hbm.at[gidx.at[bb]],
                                      rows.at[bb], sem_g[bb]).wait()
                pltpu.make_async_copy(eenc_hbm.at[pl.ds(eb + j * C, C)],
                                      embuf.at[bb], sem_e[bb]).wait()

                himask = jnp.full((16,), -65536, jnp.int32)

                def mul_body(i, _):
                    for jj in range(D // 32):
                        ew = embuf[bb, i, pl.ds(jj * 16, 16)]
                        elo = lax.bitcast_convert_type(ew << 16, jnp.float32)
                        ehi = lax.bitcast_convert_type(ew & himask, jnp.float32)
                        sl_lo = pl.ds(jj * 32, 16)
                        sl_hi = pl.ds(jj * 32 + 16, 16)
                        rows[bb, i, sl_lo] = rows[bb, i, sl_lo] * elo
                        rows[bb, i, sl_hi] = rows[bb, i, sl_hi] * ehi
                    return 0
                lax.fori_loop(0, C, mul_body, 0)
                pltpu.sync_copy(rows.at[bb], agg.at[didx.at[bb]], add=True)
                @pl.when(j + 2 < NCHUNK)
                def _():
                    start_idx(j + 2, bb)

            plsc.subcore_barrier()

            start_idx(0, 0)
            start_ge(0, 0)
            start_idx(1, 1)

            def pair_body(jj, _):
                for ph in range(NBUF):
                    phase(jj * NBUF + ph, ph)
                return 0
            lax.fori_loop(0, NCHUNK // NBUF, pair_body, 0)
            for j in range(NCHUNK - NCHUNK % NBUF, NCHUNK):
                phase(jnp.int32(j), j % NBUF)

            plsc.subcore_barrier()
            pltpu.sync_copy(agg.at[pl.ds(r0, RPT)],
                            out_hbm.at[pl.ds(b * N + r0, RPT)])
            @pl.when(s == NTILE - 1)
            def _():
                pltpu.sync_copy(agg.at[pl.ds(NTILE * RPT, TAIL)],
                                out_hbm.at[pl.ds(b * N + NTILE * RPT, TAIL)])
            if g == 0:
                zero_agg()

    return k(xenc, ei_flat, eenc)


_Q = np.concatenate([np.concatenate([np.arange(16 * j, 16 * j + 16),
                                     64 + np.arange(16 * j, 16 * j + 16)])
                     for j in range(D // 32)])


def kernel(x, edge_index, edge_attr, W_node, b_node, W_edge, b_edge, W_out, b_out):
    xenc = _linear(x.reshape(B * N, DN), W_node[:, _Q], b_node[_Q])
    eenc_i = _linear_pack(edge_attr.reshape(B * E, DE), W_edge, b_edge)
    agg = _sc_message_pass(xenc, edge_index.reshape(-1), eenc_i)
    out = _linear(agg, W_out[_Q, :], b_out, relu=True)
    return out.reshape(B, N, D)
